# baseline (speedup 1.0000x reference)
import functools

import numpy as np
import jax
import jax.numpy as jnp
from jax import lax
from jax.experimental import pallas as pl
from jax.experimental.pallas import tpu as pltpu
from jax.experimental.pallas import tpu_sc as plsc

NP_N = 10000
EP_E = 320000
NL_N = 5000
EL_E = 160000
GP_G = 100
GL_G = 49
D_IN = 128
K_HOP = 8
SEQ_L = 150
DIM_C = 45

NCORE = 2
NSUB = 16
CHUNK = 128
ZROWS = 16

NP_PAD = 10240
NL_PAD = 5120
NM_PAD = NP_PAD + NL_PAD
EP_PAD = 327680
EL_PAD = 163840
EM_PAD = 491520


def _sc_scatter_partials(tab, src1d, dst2d, n_pad, d, nbuf, gb):
    chunks = dst2d.shape[0]
    nw = NCORE * NSUB
    cpt = chunks // nw
    ngroups = cpt // (nbuf * gb)
    rows_per_tile = n_pad // NSUB
    mesh = plsc.VectorSubcoreMesh(
        core_axis_name="c", subcore_axis_name="s",
        num_cores=NCORE, num_subcores=NSUB)

    @functools.partial(
        pl.kernel,
        out_type=jax.ShapeDtypeStruct((NCORE * n_pad, d), jnp.float32),
        mesh=mesh,
        compiler_params=pltpu.CompilerParams(use_tc_tiling_on_sc=False),
        scratch_types=[
            pltpu.VMEM((nbuf, gb * CHUNK), jnp.int32),
            pltpu.VMEM((nbuf, gb, CHUNK), jnp.int32),
            pltpu.VMEM((nbuf, gb * CHUNK, d), jnp.float32),
            pltpu.VMEM((ZROWS, d), jnp.float32),
            pltpu.VMEM_SHARED((n_pad, d), jnp.float32),
            pltpu.SemaphoreType.DMA,
            pltpu.SemaphoreType.DMA,
            pltpu.SemaphoreType.DMA,
        ],
    )
    def k(tab_hbm, src_hbm, dst_hbm, out_hbm,
          src_v, dst_v, bufs, zero_v, acc_sh, isem, gsem, ssem):
        cid = lax.axis_index("c")
        sid = lax.axis_index("s")
        tile = cid * NSUB + sid
        e0 = tile * cpt * CHUNK
        c0 = tile * cpt

        for i in range(ZROWS):
            for j in range(d // 16):
                zero_v[i, pl.ds(j * 16, 16)] = jnp.zeros((16,), jnp.float32)
        row0 = sid * rows_per_tile

        def zbody(i, carry):
            pltpu.async_copy(
                zero_v, acc_sh.at[pl.ds(row0 + i * ZROWS, ZROWS)], isem)
            return carry
        lax.fori_loop(0, rows_per_tile // ZROWS, zbody, 0)

        def zwait(i, carry):
            pltpu.make_async_copy(
                zero_v, acc_sh.at[pl.ds(row0 + i * ZROWS, ZROWS)], isem).wait()
            return carry
        lax.fori_loop(0, rows_per_tile // ZROWS, zwait, 0)
        plsc.subcore_barrier()

        def idx_copy(g, b, start):
            gc = (g * nbuf + b) * gb
            s_hbm = src_hbm.at[pl.ds(e0 + gc * CHUNK, gb * CHUNK)]
            d_hbm = dst_hbm.at[pl.ds(c0 + gc, gb)]
            if start:
                pltpu.async_copy(s_hbm, src_v.at[b], isem)
                pltpu.async_copy(d_hbm, dst_v.at[b], isem)
            else:
                pltpu.make_async_copy(s_hbm, src_v.at[b], isem).wait()
                pltpu.make_async_copy(d_hbm, dst_v.at[b], isem).wait()

        for b in range(nbuf):
            idx_copy(0, b, True)
        for b in range(nbuf):
            idx_copy(0, b, False)
            pltpu.async_copy(tab_hbm.at[src_v.at[b]], bufs.at[b], gsem)

        def gbody(g, carry):
            for b in range(nbuf):
                pltpu.make_async_copy(
                    tab_hbm.at[src_v.at[b]], bufs.at[b], gsem).wait()
                for j in range(gb):
                    pltpu.async_copy(
                        bufs.at[b, pl.ds(j * CHUNK, CHUNK)],
                        acc_sh.at[dst_v.at[b, j]], ssem, add=True)
            for b in range(nbuf):
                for j in range(gb):
                    pltpu.make_async_copy(
                        bufs.at[b, pl.ds(j * CHUNK, CHUNK)],
                        acc_sh.at[dst_v.at[b, j]], ssem).wait()

                @pl.when(g < ngroups - 1)
                def _regather(b=b):
                    idx_copy(g + 1, b, True)
                    idx_copy(g + 1, b, False)
                    pltpu.async_copy(
                        tab_hbm.at[src_v.at[b]], bufs.at[b], gsem)
            return carry
        lax.fori_loop(0, ngroups, gbody, 0)
        plsc.subcore_barrier()

        out0 = cid * n_pad + row0
        pltpu.sync_copy(acc_sh.at[pl.ds(row0, rows_per_tile)],
                        out_hbm.at[pl.ds(out0, rows_per_tile)])

    return k(tab, src1d, dst2d)


def _sc_tag_layer(h0, norm16, src1d, dst2d, n_pad, d, nbuf, gb):
    chunks = dst2d.shape[0]
    cpt = chunks // NSUB
    ngroups = cpt // (nbuf * gb)
    rows_per_tile = n_pad // NSUB
    rb = 32
    mesh = plsc.VectorSubcoreMesh(
        core_axis_name="c", subcore_axis_name="s",
        num_cores=NCORE, num_subcores=NSUB)

    @functools.partial(
        pl.kernel,
        out_type=[jax.ShapeDtypeStruct((K_HOP * n_pad, d), jnp.float32),
                  jax.ShapeDtypeStruct((NCORE * n_pad, d), jnp.float32)],
        mesh=mesh,
        compiler_params=pltpu.CompilerParams(use_tc_tiling_on_sc=False),
        scratch_types=[
            pltpu.VMEM((nbuf, gb * CHUNK), jnp.int32),
            pltpu.VMEM((nbuf, gb, CHUNK), jnp.int32),
            pltpu.VMEM((nbuf, gb * CHUNK, d), jnp.float32),
            pltpu.VMEM((ZROWS, d), jnp.float32),
            pltpu.VMEM((rows_per_tile, 16), jnp.float32),
            pltpu.VMEM((rb, d), jnp.float32),
            pltpu.VMEM((rb, d), jnp.float32),
            pltpu.VMEM_SHARED((n_pad, d), jnp.float32),
            pltpu.SemaphoreType.DMA,
            pltpu.SemaphoreType.DMA,
            pltpu.SemaphoreType.DMA,
        ],
    )
    def k(h0_hbm, nrm_hbm, src_hbm, dst_hbm, feats_hbm, s2_hbm,
          src_v, dst_v, bufs, zero_v, nrm_v, abuf, hbuf, acc_sh,
          isem, gsem, ssem):
        cid = lax.axis_index("c")
        sid = lax.axis_index("s")
        e0 = sid * cpt * CHUNK
        c0 = sid * cpt
        row0 = sid * rows_per_tile
        soff = cid * n_pad

        pltpu.sync_copy(nrm_hbm.at[pl.ds(row0, rows_per_tile)], nrm_v)
        for i in range(ZROWS):
            for j in range(d // 16):
                zero_v[i, pl.ds(j * 16, 16)] = jnp.zeros((16,), jnp.float32)

        def scale_rows(i, out_h):
            def rowf(j, carry):
                nv = nrm_v[i * rb + j, pl.ds(0, 16)]
                for jv in range(d // 16):
                    a = abuf[j, pl.ds(jv * 16, 16)]
                    h = a * nv
                    if out_h:
                        abuf[j, pl.ds(jv * 16, 16)] = h
                    hbuf[j, pl.ds(jv * 16, 16)] = h * nv
                return carry
            lax.fori_loop(0, rb, rowf, 0)

        def p0(i, carry):
            pltpu.sync_copy(h0_hbm.at[pl.ds(row0 + i * rb, rb)], abuf)
            scale_rows(i, False)
            pltpu.sync_copy(hbuf, s2_hbm.at[pl.ds(soff + row0 + i * rb, rb)])
            return carry
        lax.fori_loop(0, rows_per_tile // rb, p0, 0)
        plsc.subcore_barrier()

        def add_soff(b):
            for v in range(gb * CHUNK // 16):
                src_v[b, pl.ds(v * 16, 16)] = \
                    src_v[b, pl.ds(v * 16, 16)] + soff

        def idx_copy(g, b, start):
            gc = (g * nbuf + b) * gb
            s_hbm = src_hbm.at[pl.ds(e0 + gc * CHUNK, gb * CHUNK)]
            d_hbm = dst_hbm.at[pl.ds(c0 + gc, gb)]
            if start:
                pltpu.async_copy(s_hbm, src_v.at[b], isem)
                pltpu.async_copy(d_hbm, dst_v.at[b], isem)
            else:
                pltpu.make_async_copy(s_hbm, src_v.at[b], isem).wait()
                pltpu.make_async_copy(d_hbm, dst_v.at[b], isem).wait()

        def hop(kh, carry):
            def zbody(i, c2):
                pltpu.async_copy(
                    zero_v, acc_sh.at[pl.ds(row0 + i * ZROWS, ZROWS)], isem)
                return c2
            lax.fori_loop(0, rows_per_tile // ZROWS, zbody, 0)

            def zwait(i, c2):
                pltpu.make_async_copy(
                    zero_v,
                    acc_sh.at[pl.ds(row0 + i * ZROWS, ZROWS)], isem).wait()
                return c2
            lax.fori_loop(0, rows_per_tile // ZROWS, zwait, 0)
            plsc.subcore_barrier()

            for b in range(nbuf):
                idx_copy(0, b, True)
            for b in range(nbuf):
                idx_copy(0, b, False)
                add_soff(b)
                pltpu.async_copy(s2_hbm.at[src_v.at[b]], bufs.at[b], gsem)

            def gbody(g, c2):
                for b in range(nbuf):
                    pltpu.make_async_copy(
                        s2_hbm.at[src_v.at[b]], bufs.at[b], gsem).wait()
                    for j in range(gb):
                        pltpu.async_copy(
                            bufs.at[b, pl.ds(j * CHUNK, CHUNK)],
                            acc_sh.at[dst_v.at[b, j]], ssem, add=True)
                for b in range(nbuf):
                    for j in range(gb):
                        pltpu.make_async_copy(
                            bufs.at[b, pl.ds(j * CHUNK, CHUNK)],
                            acc_sh.at[dst_v.at[b, j]], ssem).wait()

                    @pl.when(g < ngroups - 1)
                    def _regather(b=b):
                        idx_copy(g + 1, b, True)
                        idx_copy(g + 1, b, False)
                        add_soff(b)
                        pltpu.async_copy(
                            s2_hbm.at[src_v.at[b]], bufs.at[b], gsem)
                return c2
            lax.fori_loop(0, ngroups, gbody, 0)
            plsc.subcore_barrier()

            def cb(i, c2):
                pltpu.sync_copy(acc_sh.at[pl.ds(row0 + i * rb, rb)], abuf)
                scale_rows(i, True)
                pltpu.sync_copy(
                    abuf,
                    feats_hbm.at[pl.ds(kh * n_pad + row0 + i * rb, rb)])
                pltpu.sync_copy(
                    hbuf, s2_hbm.at[pl.ds(soff + row0 + i * rb, rb)])
                return c2
            lax.fori_loop(0, rows_per_tile // rb, cb, 0)
            plsc.subcore_barrier()
            return carry
        lax.fori_loop(0, K_HOP, hop, 0)

    return k(h0, norm16, src1d, dst2d)[0]


def _tc_norm_from_deg(p):
    n_pad = p.shape[1]

    def body(p_ref, o_ref):
        deg = p_ref[0] + p_ref[1]
        o_ref[...] = lax.rsqrt(jnp.maximum(deg, 1.0))

    return pl.pallas_call(
        body, out_shape=jax.ShapeDtypeStruct((n_pad, 16), jnp.float32))(p)


def _tc_scale(xs, norm16):
    ng = len(xs)
    n_pad, d = xs[0].shape
    BN = 1024

    def body(*refs):
        nrm = refs[ng][...][:, :1]
        for gi in range(ng):
            refs[ng + 1 + gi][...] = refs[gi][...] * nrm

    spec = pl.BlockSpec((BN, d), lambda i: (i, 0))
    return pl.pallas_call(
        body, grid=(n_pad // BN,),
        in_specs=[spec] * ng + [pl.BlockSpec((BN, 16), lambda i: (i, 0))],
        out_specs=[spec] * ng,
        out_shape=[jax.ShapeDtypeStruct((n_pad, d), jnp.float32)] * ng)(
            *xs, norm16)


def _tc_combine(ps, norm16):
    ng = len(ps)
    _, n_pad, d = ps[0].shape
    BN = 1024

    def body(*refs):
        nrm = refs[ng][...][:, :1]
        for gi in range(ng):
            h = (refs[gi][0] + refs[gi][1]) * nrm
            refs[ng + 1 + gi][...] = h
            refs[ng + 1 + ng + gi][...] = h * nrm

    pspec = pl.BlockSpec((2, BN, d), lambda i: (0, i, 0))
    ospec = pl.BlockSpec((BN, d), lambda i: (i, 0))
    outs = pl.pallas_call(
        body, grid=(n_pad // BN,),
        in_specs=[pspec] * ng + [pl.BlockSpec((BN, 16), lambda i: (i, 0))],
        out_specs=[ospec] * (2 * ng),
        out_shape=[jax.ShapeDtypeStruct((n_pad, d), jnp.float32)] * (2 * ng))(
            *ps, norm16)
    return list(outs[:ng]), list(outs[ng:])


def _tc_tag_matmul(feats, w_pad, b_pad, blk_off, n_rows):
    k1 = len(feats)
    d_in_p = feats[0].shape[1]
    d_out_p = w_pad.shape[2]
    BN = 1024

    def body(*refs):
        f_refs = refs[:k1]
        w_ref, b_ref, h_ref = refs[k1], refs[k1 + 1], refs[k1 + 2]
        acc = jnp.zeros((BN, d_out_p), jnp.float32)
        for k in range(k1):
            acc = acc + jnp.dot(f_refs[k][...], w_ref[k],
                                preferred_element_type=jnp.float32)
        y = jnp.maximum(acc + b_ref[...], 0.0)
        ss = jnp.sum(y * y, axis=1, keepdims=True)
        h_ref[...] = y / jnp.maximum(jnp.sqrt(ss), 1e-12)

    in_specs = [pl.BlockSpec((BN, d_in_p), lambda i, o=blk_off: (i + o, 0))
                for _ in range(k1)]
    in_specs += [pl.BlockSpec((k1, d_in_p, d_out_p), lambda i: (0, 0, 0)),
                 pl.BlockSpec((1, d_out_p), lambda i: (0, 0))]
    return pl.pallas_call(
        body, grid=(n_rows // BN,),
        in_specs=in_specs,
        out_specs=pl.BlockSpec((BN, d_out_p), lambda i: (i, 0)),
        out_shape=jax.ShapeDtypeStruct((n_rows, d_out_p), jnp.float32))(
            *feats, w_pad, b_pad)


def _tc_segment_max(h, ids, n_seg):
    n_pad, d = h.shape
    g_pad = (n_seg + 7) // 8 * 8

    def body(h_ref, id_ref, o_ref):
        def gbody(g, carry):
            m = id_ref[...] == g
            vals = jnp.where(m, h_ref[...], -jnp.inf)
            mx = jnp.max(vals, axis=0, keepdims=True)
            o_ref[pl.ds(g, 1), :] = jnp.where(jnp.isfinite(mx), mx, 0.0)
            return carry
        lax.fori_loop(0, n_seg, gbody, 0)

    return pl.pallas_call(
        body, out_shape=jax.ShapeDtypeStruct((g_pad, d), jnp.float32))(h, ids)


def _tc_attention(seq, mask, wqkv, bqkv, wproj, bproj):
    def body(x_ref, m_ref, wq_ref, bq_ref, wp_ref, bp_ref, o_ref):
        x = x_ref[...]
        qkv = jnp.dot(x, wq_ref[...], preferred_element_type=jnp.float32)
        qkv = qkv + bq_ref[...]
        q = qkv[:, :DIM_C]
        kk = qkv[:, DIM_C:2 * DIM_C]
        v = qkv[:, 2 * DIM_C:]
        a = lax.dot_general(q, kk, (((1,), (1,)), ((), ())),
                            preferred_element_type=jnp.float32)
        a = a * (DIM_C ** -0.5)
        a = jnp.where(m_ref[...] == 0.0, -1e9, a)
        a = a - jnp.max(a, axis=1, keepdims=True)
        e = jnp.exp(a)
        p = e / jnp.sum(e, axis=1, keepdims=True)
        o = jnp.dot(p, v, preferred_element_type=jnp.float32)
        o_ref[...] = jnp.dot(o, wp_ref[...],
                             preferred_element_type=jnp.float32) + bp_ref[...]

    return pl.pallas_call(
        body, out_shape=jax.ShapeDtypeStruct((SEQ_L, DIM_C), jnp.float32))(
            seq, mask, wqkv, bqkv.reshape(1, -1), wproj, bproj.reshape(1, -1))


def _tc_mlp1(x, w, b):
    kp, n_out = w.shape
    BK = 512

    def body(x_ref, w_ref, b_ref, o_ref):
        @pl.when(pl.program_id(0) == 0)
        def _init():
            o_ref[...] = jnp.zeros_like(o_ref)

        o_ref[...] += jnp.dot(x_ref[...], w_ref[...],
                              preferred_element_type=jnp.float32)

        @pl.when(pl.program_id(0) == pl.num_programs(0) - 1)
        def _fin():
            o_ref[...] = jnp.maximum(o_ref[...] + b_ref[...], 0.0)

    return pl.pallas_call(
        body, grid=(kp // BK,),
        in_specs=[pl.BlockSpec((1, BK), lambda i: (0, i)),
                  pl.BlockSpec((BK, n_out), lambda i: (i, 0)),
                  pl.BlockSpec((1, n_out), lambda i: (0, 0))],
        out_specs=pl.BlockSpec((1, n_out), lambda i: (0, 0)),
        out_shape=jax.ShapeDtypeStruct((1, n_out), jnp.float32))(x, w, b)


def _tc_mlp_rest(x, w2, b2, w3, b3, w4, b4):
    def body(x_ref, w2_ref, b2_ref, w3_ref, b3_ref, w4_ref, b4_ref, o_ref):
        h = jnp.dot(x_ref[...], w2_ref[...], preferred_element_type=jnp.float32)
        h = jnp.maximum(h + b2_ref[...], 0.0)
        h = jnp.dot(h, w3_ref[...], preferred_element_type=jnp.float32)
        h = jnp.maximum(h + b3_ref[...], 0.0)
        z = jnp.dot(h, w4_ref[...], preferred_element_type=jnp.float32)
        z = z + b4_ref[...]
        o_ref[...] = 1.0 / (1.0 + jnp.exp(-z))

    return pl.pallas_call(
        body, out_shape=jax.ShapeDtypeStruct((1, w4.shape[1]), jnp.float32))(
            x, w2, b2, w3, b3, w4, b4)


def _attn_mask_np():
    n = GL_G + GP_G
    m = np.eye(SEQ_L, dtype=np.float32)
    m[n:, :] = 0.0
    m[:, n:] = 0.0
    m[:, n - 1] = 1.0
    m[n - 1, :] = 1.0
    m[n - 1, n - 1] = 0.0
    return m


def _pad2(x, r, c):
    out = jnp.zeros((r, c), jnp.float32)
    return out.at[:x.shape[0], :x.shape[1]].set(x)


def _pad_w(w, b, d_in, d_in_p, d_out, d_out_p, ng=1):
    dg = d_in // ng
    w_r = w.reshape((K_HOP + 1) * ng, dg, d_out)
    w_pad = jnp.zeros(((K_HOP + 1) * ng, d_in_p, d_out_p), jnp.float32)
    w_pad = w_pad.at[:, :dg, :d_out].set(w_r)
    b_pad = jnp.zeros((1, d_out_p), jnp.float32).at[0, :d_out].set(b)
    return w_pad, b_pad


def _pad_edges(src, dst, e_pad, zero_row):
    npad = e_pad - src.shape[0]
    src_p = jnp.concatenate([src, jnp.full((npad,), zero_row, jnp.int32)])
    dst_p = jnp.concatenate([dst, jnp.full((npad,), zero_row, jnp.int32)])
    return src_p, dst_p.reshape(-1, CHUNK)


def _hops(h0, norm16, src1d, dst2d, n_pad, d, nbuf, gb):
    feats_flat = _sc_tag_layer(h0, norm16, src1d, dst2d, n_pad, d, nbuf, gb)
    f3 = feats_flat.reshape(K_HOP, n_pad, d)
    return [h0] + [f3[kk] for kk in range(K_HOP)]


def kernel(x_protein, x_ligand, edge_index_protein, edge_index_ligand,
           graph_ids_protein, graph_ids_ligand, Wp1, bp1, Wp2, bp2,
           Wl1, bl1, Wl2, bl2, Wl3, bl3, Wqkv, bqkv, Wproj, bproj,
           Wf1, bf1, Wf2, bf2, Wf3, bf3, Wout, bout):
    srcm = jnp.concatenate([edge_index_protein[0],
                            edge_index_ligand[0] + NP_PAD])
    dstm = jnp.concatenate([edge_index_protein[1],
                            edge_index_ligand[1] + NP_PAD])
    srcm1, dstm2 = _pad_edges(srcm, dstm, EM_PAD, NM_PAD - 1)
    srcp1, dstp2 = _pad_edges(edge_index_protein[0], edge_index_protein[1],
                              EP_PAD, NP_PAD - 1)
    srcl1, dstl2 = _pad_edges(edge_index_ligand[0], edge_index_ligand[1],
                              EL_PAD, NL_PAD - 1)

    ones_tab = (jnp.zeros((NM_PAD, 16), jnp.float32)
                .at[:NP_N].set(1.0)
                .at[NP_PAD:NP_PAD + NL_N].set(1.0))
    degp = _sc_scatter_partials(ones_tab, srcm1, dstm2, NM_PAD, 16, 2, 4)
    norm16 = _tc_norm_from_deg(degp.reshape(2, NM_PAD, 16))
    normp16 = norm16[:NP_PAD]
    norml16 = norm16[NP_PAD:]

    xp = jnp.zeros((NP_PAD, D_IN), jnp.float32).at[:NP_N].set(x_protein)
    xl = jnp.zeros((NL_PAD, D_IN), jnp.float32).at[:NL_N].set(x_ligand)

    feats_p = _hops(xp, normp16, srcp1, dstp2, NP_PAD, 128, 1, 1)
    feats_l = _hops(xl, norml16, srcl1, dstl2, NL_PAD, 128, 1, 1)
    wpp, bpp = _pad_w(Wp1, bp1, 128, 128, 50, 64)
    wlp, blp = _pad_w(Wl1, bl1, 128, 128, 50, 64)
    hp = _tc_tag_matmul(feats_p, wpp, bpp, 0, NP_PAD)
    hl = _tc_tag_matmul(feats_l, wlp, blp, 0, NL_PAD)
    h = jnp.concatenate([hp, hl], axis=0)

    feats = _hops(h, norm16, srcm1, dstm2, NM_PAD, 64, 2, 2)
    wpp, bpp = _pad_w(Wp2, bp2, 50, 64, 45, 48)
    wlp, blp = _pad_w(Wl2, bl2, 50, 64, 45, 48)
    hp = _tc_tag_matmul(feats, wpp, bpp, 0, NP_PAD)
    hl = _tc_tag_matmul(feats, wlp, blp, NP_PAD // 1024, NL_PAD)

    feats = _hops(hl, norml16, srcl1, dstl2, NL_PAD, 48, 2, 2)
    wlp, blp = _pad_w(Wl3, bl3, 45, 48, 45, 48)
    hl = _tc_tag_matmul(feats, wlp, blp, 0, NL_PAD)

    ids_p = jnp.concatenate(
        [graph_ids_protein,
         jnp.full((NP_PAD - NP_N,), np.int32(10 ** 6), jnp.int32)])
    ids_l = jnp.concatenate(
        [graph_ids_ligand,
         jnp.full((NL_PAD - NL_N,), np.int32(10 ** 6), jnp.int32)])
    prot_rep = _tc_segment_max(hp, ids_p.reshape(NP_PAD, 1), GP_G)
    lig_rep = _tc_segment_max(hl, ids_l.reshape(NL_PAD, 1), GL_G)

    seq = jnp.concatenate(
        [lig_rep[:GL_G, :DIM_C], prot_rep[:GP_G, :DIM_C],
         jnp.zeros((SEQ_L - GL_G - GP_G, DIM_C), jnp.float32)], axis=0)
    mask = jnp.asarray(_attn_mask_np())
    att = _tc_attention(seq, mask, Wqkv, bqkv, Wproj, bproj)

    xh = att.reshape(1, SEQ_L * DIM_C)
    xh_p = _pad2(xh, 1, 7168)
    w1 = _pad2(Wf1, 7168, 2048)
    b1 = _pad2(bf1.reshape(1, -1), 1, 2048)
    h1 = _tc_mlp1(xh_p, w1, b1)

    w2 = _pad2(Wf2, 2048, 1024)
    b2 = _pad2(bf2.reshape(1, -1), 1, 1024)
    w3 = _pad2(Wf3, 1024, 512)
    b3 = _pad2(bf3.reshape(1, -1), 1, 512)
    w4 = _pad2(Wout, 512, 128)
    b4 = _pad2(bout.reshape(1, -1), 1, 128)
    out = _tc_mlp_rest(h1, w2, b2, w3, b3, w4, b4)
    return out[0:1, 0:1]


# speedup vs baseline: 1.9512x; 1.9512x over previous
import functools

import numpy as np
import jax
import jax.numpy as jnp
from jax import lax
from jax.experimental import pallas as pl
from jax.experimental.pallas import tpu as pltpu
from jax.experimental.pallas import tpu_sc as plsc

NP_N = 10000
EP_E = 320000
NL_N = 5000
EL_E = 160000
GP_G = 100
GL_G = 49
D_IN = 128
K_HOP = 8
SEQ_L = 150
DIM_C = 45

NCORE = 2
NSUB = 16
CHUNK = 128
ZROWS = 16

NP_PAD = 10240
NL_PAD = 5120
NM_PAD = NP_PAD + NL_PAD
EP_PAD = 327680
EL_PAD = 163840
EM_PAD = 491520


def _sc_scatter_partials(tab, src1d, dst2d, n_pad, d, nbuf, gb):
    chunks = dst2d.shape[0]
    nw = NCORE * NSUB
    cpt = chunks // nw
    ngroups = cpt // (nbuf * gb)
    rows_per_tile = n_pad // NSUB
    mesh = plsc.VectorSubcoreMesh(
        core_axis_name="c", subcore_axis_name="s",
        num_cores=NCORE, num_subcores=NSUB)

    @functools.partial(
        pl.kernel,
        out_type=jax.ShapeDtypeStruct((NCORE * n_pad, d), jnp.float32),
        mesh=mesh,
        compiler_params=pltpu.CompilerParams(use_tc_tiling_on_sc=False),
        scratch_types=[
            pltpu.VMEM((nbuf, gb * CHUNK), jnp.int32),
            pltpu.VMEM((nbuf, gb, CHUNK), jnp.int32),
            pltpu.VMEM((nbuf, gb * CHUNK, d), jnp.float32),
            pltpu.VMEM((ZROWS, d), jnp.float32),
            pltpu.VMEM_SHARED((n_pad, d), jnp.float32),
            pltpu.SemaphoreType.DMA,
            pltpu.SemaphoreType.DMA,
            pltpu.SemaphoreType.DMA,
        ],
    )
    def k(tab_hbm, src_hbm, dst_hbm, out_hbm,
          src_v, dst_v, bufs, zero_v, acc_sh, isem, gsem, ssem):
        cid = lax.axis_index("c")
        sid = lax.axis_index("s")
        tile = cid * NSUB + sid
        e0 = tile * cpt * CHUNK
        c0 = tile * cpt

        for i in range(ZROWS):
            for j in range(d // 16):
                zero_v[i, pl.ds(j * 16, 16)] = jnp.zeros((16,), jnp.float32)
        row0 = sid * rows_per_tile

        def zbody(i, carry):
            pltpu.async_copy(
                zero_v, acc_sh.at[pl.ds(row0 + i * ZROWS, ZROWS)], isem)
            return carry
        lax.fori_loop(0, rows_per_tile // ZROWS, zbody, 0)

        def zwait(i, carry):
            pltpu.make_async_copy(
                zero_v, acc_sh.at[pl.ds(row0 + i * ZROWS, ZROWS)], isem).wait()
            return carry
        lax.fori_loop(0, rows_per_tile // ZROWS, zwait, 0)
        plsc.subcore_barrier()

        def idx_copy(g, b, start):
            gc = (g * nbuf + b) * gb
            s_hbm = src_hbm.at[pl.ds(e0 + gc * CHUNK, gb * CHUNK)]
            d_hbm = dst_hbm.at[pl.ds(c0 + gc, gb)]
            if start:
                pltpu.async_copy(s_hbm, src_v.at[b], isem)
                pltpu.async_copy(d_hbm, dst_v.at[b], isem)
            else:
                pltpu.make_async_copy(s_hbm, src_v.at[b], isem).wait()
                pltpu.make_async_copy(d_hbm, dst_v.at[b], isem).wait()

        for b in range(nbuf):
            idx_copy(0, b, True)
        for b in range(nbuf):
            idx_copy(0, b, False)
            pltpu.async_copy(tab_hbm.at[src_v.at[b]], bufs.at[b], gsem)

        def gbody(g, carry):
            for b in range(nbuf):
                pltpu.make_async_copy(
                    tab_hbm.at[src_v.at[b]], bufs.at[b], gsem).wait()
                for j in range(gb):
                    pltpu.async_copy(
                        bufs.at[b, pl.ds(j * CHUNK, CHUNK)],
                        acc_sh.at[dst_v.at[b, j]], ssem, add=True)
            for b in range(nbuf):
                for j in range(gb):
                    pltpu.make_async_copy(
                        bufs.at[b, pl.ds(j * CHUNK, CHUNK)],
                        acc_sh.at[dst_v.at[b, j]], ssem).wait()

                @pl.when(g < ngroups - 1)
                def _regather(b=b):
                    idx_copy(g + 1, b, True)
                    idx_copy(g + 1, b, False)
                    pltpu.async_copy(
                        tab_hbm.at[src_v.at[b]], bufs.at[b], gsem)
            return carry
        lax.fori_loop(0, ngroups, gbody, 0)
        plsc.subcore_barrier()

        out0 = cid * n_pad + row0
        pltpu.sync_copy(acc_sh.at[pl.ds(row0, rows_per_tile)],
                        out_hbm.at[pl.ds(out0, rows_per_tile)])

    return k(tab, src1d, dst2d)


def _sc_tag_layer(h0, norm16, src1d, dst2d, n_pad, d, nbuf, gb):
    chunks = dst2d.shape[0]
    cpt = chunks // NSUB
    ngroups = cpt // (nbuf * gb)
    rows_per_tile = n_pad // NSUB
    rb = 32
    mesh = plsc.VectorSubcoreMesh(
        core_axis_name="c", subcore_axis_name="s",
        num_cores=NCORE, num_subcores=NSUB)

    @functools.partial(
        pl.kernel,
        out_type=[jax.ShapeDtypeStruct((NCORE * K_HOP * n_pad, d),
                                       jnp.float32),
                  jax.ShapeDtypeStruct((NCORE * n_pad, d), jnp.float32)],
        mesh=mesh,
        compiler_params=pltpu.CompilerParams(use_tc_tiling_on_sc=False),
        scratch_types=[
            pltpu.VMEM((nbuf, gb * CHUNK), jnp.int32),
            pltpu.VMEM((nbuf, gb, CHUNK), jnp.int32),
            pltpu.VMEM((nbuf, gb * CHUNK, d), jnp.float32),
            pltpu.VMEM((ZROWS, d), jnp.float32),
            pltpu.VMEM((rows_per_tile, 16), jnp.float32),
            pltpu.VMEM((rb, d), jnp.float32),
            pltpu.VMEM((rb, d), jnp.float32),
            pltpu.VMEM_SHARED((n_pad, d), jnp.float32),
            pltpu.SemaphoreType.DMA,
            pltpu.SemaphoreType.DMA,
            pltpu.SemaphoreType.DMA,
        ],
    )
    def k(h0_hbm, nrm_hbm, src_hbm, dst_hbm, feats_hbm, s2_hbm,
          src_v, dst_v, bufs, zero_v, nrm_v, abuf, hbuf, acc_sh,
          isem, gsem, ssem):
        cid = lax.axis_index("c")
        sid = lax.axis_index("s")
        e0 = sid * cpt * CHUNK
        c0 = sid * cpt
        row0 = sid * rows_per_tile
        soff = cid * n_pad

        pltpu.sync_copy(nrm_hbm.at[pl.ds(row0, rows_per_tile)], nrm_v)
        for i in range(ZROWS):
            for j in range(d // 16):
                zero_v[i, pl.ds(j * 16, 16)] = jnp.zeros((16,), jnp.float32)

        def scale_rows(i, out_h):
            def rowf(j, carry):
                nv = nrm_v[i * rb + j, pl.ds(0, 16)]
                for jv in range(d // 16):
                    a = abuf[j, pl.ds(jv * 16, 16)]
                    h = a * nv
                    if out_h:
                        abuf[j, pl.ds(jv * 16, 16)] = h
                    hbuf[j, pl.ds(jv * 16, 16)] = h * nv
                return carry
            lax.fori_loop(0, rb, rowf, 0)

        def p0(i, carry):
            pltpu.sync_copy(h0_hbm.at[pl.ds(soff + row0 + i * rb, rb)], abuf)
            scale_rows(i, False)
            pltpu.sync_copy(hbuf, s2_hbm.at[pl.ds(soff + row0 + i * rb, rb)])
            return carry
        lax.fori_loop(0, rows_per_tile // rb, p0, 0)
        plsc.subcore_barrier()

        def add_soff(b):
            for v in range(gb * CHUNK // 16):
                src_v[b, pl.ds(v * 16, 16)] = \
                    src_v[b, pl.ds(v * 16, 16)] + soff

        def idx_copy(g, b, start):
            gc = (g * nbuf + b) * gb
            s_hbm = src_hbm.at[pl.ds(e0 + gc * CHUNK, gb * CHUNK)]
            d_hbm = dst_hbm.at[pl.ds(c0 + gc, gb)]
            if start:
                pltpu.async_copy(s_hbm, src_v.at[b], isem)
                pltpu.async_copy(d_hbm, dst_v.at[b], isem)
            else:
                pltpu.make_async_copy(s_hbm, src_v.at[b], isem).wait()
                pltpu.make_async_copy(d_hbm, dst_v.at[b], isem).wait()

        def hop(kh, carry):
            def zbody(i, c2):
                pltpu.async_copy(
                    zero_v, acc_sh.at[pl.ds(row0 + i * ZROWS, ZROWS)], isem)
                return c2
            lax.fori_loop(0, rows_per_tile // ZROWS, zbody, 0)

            def zwait(i, c2):
                pltpu.make_async_copy(
                    zero_v,
                    acc_sh.at[pl.ds(row0 + i * ZROWS, ZROWS)], isem).wait()
                return c2
            lax.fori_loop(0, rows_per_tile // ZROWS, zwait, 0)
            plsc.subcore_barrier()

            for b in range(nbuf):
                idx_copy(0, b, True)
            for b in range(nbuf):
                idx_copy(0, b, False)
                add_soff(b)
                pltpu.async_copy(s2_hbm.at[src_v.at[b]], bufs.at[b], gsem)

            def gbody(g, c2):
                for b in range(nbuf):
                    pltpu.make_async_copy(
                        s2_hbm.at[src_v.at[b]], bufs.at[b], gsem).wait()
                    for j in range(gb):
                        pltpu.async_copy(
                            bufs.at[b, pl.ds(j * CHUNK, CHUNK)],
                            acc_sh.at[dst_v.at[b, j]], ssem, add=True)
                for b in range(nbuf):
                    for j in range(gb):
                        pltpu.make_async_copy(
                            bufs.at[b, pl.ds(j * CHUNK, CHUNK)],
                            acc_sh.at[dst_v.at[b, j]], ssem).wait()

                    @pl.when(g < ngroups - 1)
                    def _regather(b=b):
                        idx_copy(g + 1, b, True)
                        idx_copy(g + 1, b, False)
                        add_soff(b)
                        pltpu.async_copy(
                            s2_hbm.at[src_v.at[b]], bufs.at[b], gsem)
                return c2
            lax.fori_loop(0, ngroups, gbody, 0)
            plsc.subcore_barrier()

            def cb(i, c2):
                pltpu.sync_copy(acc_sh.at[pl.ds(row0 + i * rb, rb)], abuf)
                scale_rows(i, True)
                pltpu.sync_copy(
                    abuf,
                    feats_hbm.at[pl.ds((cid * K_HOP + kh) * n_pad
                                       + row0 + i * rb, rb)])
                pltpu.sync_copy(
                    hbuf, s2_hbm.at[pl.ds(soff + row0 + i * rb, rb)])
                return c2
            lax.fori_loop(0, rows_per_tile // rb, cb, 0)
            plsc.subcore_barrier()
            return carry
        lax.fori_loop(0, K_HOP, hop, 0)

    return k(h0, norm16, src1d, dst2d)[0]


def _hops(h0, norm16, src1d, dst2d, n_pad, nbuf, gb, split):
    d = h0.shape[1]
    if split:
        dh = d // 2
        h0_in = jnp.concatenate([h0[:, :dh], h0[:, dh:]], axis=0)
        groups0 = [h0[:, :dh], h0[:, dh:]]
    else:
        dh = d
        h0_in = jnp.concatenate([h0, h0], axis=0)
        groups0 = [h0]
    f = _sc_tag_layer(h0_in, norm16, src1d, dst2d, n_pad, dh, nbuf, gb)
    f4 = f.reshape(NCORE, K_HOP, n_pad, dh)
    feats = list(groups0)
    for kk in range(K_HOP):
        if split:
            feats.extend([f4[0, kk], f4[1, kk]])
        else:
            feats.append(f4[0, kk])
    return feats


def _tc_norm_from_deg(p):
    n_pad = p.shape[1]

    def body(p_ref, o_ref):
        deg = p_ref[0] + p_ref[1]
        o_ref[...] = lax.rsqrt(jnp.maximum(deg, 1.0))

    return pl.pallas_call(
        body, out_shape=jax.ShapeDtypeStruct((n_pad, 16), jnp.float32))(p)


def _tc_scale(xs, norm16):
    ng = len(xs)
    n_pad, d = xs[0].shape
    BN = 1024

    def body(*refs):
        nrm = refs[ng][...][:, :1]
        for gi in range(ng):
            refs[ng + 1 + gi][...] = refs[gi][...] * nrm

    spec = pl.BlockSpec((BN, d), lambda i: (i, 0))
    return pl.pallas_call(
        body, grid=(n_pad // BN,),
        in_specs=[spec] * ng + [pl.BlockSpec((BN, 16), lambda i: (i, 0))],
        out_specs=[spec] * ng,
        out_shape=[jax.ShapeDtypeStruct((n_pad, d), jnp.float32)] * ng)(
            *xs, norm16)


def _tc_combine(ps, norm16):
    ng = len(ps)
    _, n_pad, d = ps[0].shape
    BN = 1024

    def body(*refs):
        nrm = refs[ng][...][:, :1]
        for gi in range(ng):
            h = (refs[gi][0] + refs[gi][1]) * nrm
            refs[ng + 1 + gi][...] = h
            refs[ng + 1 + ng + gi][...] = h * nrm

    pspec = pl.BlockSpec((2, BN, d), lambda i: (0, i, 0))
    ospec = pl.BlockSpec((BN, d), lambda i: (i, 0))
    outs = pl.pallas_call(
        body, grid=(n_pad // BN,),
        in_specs=[pspec] * ng + [pl.BlockSpec((BN, 16), lambda i: (i, 0))],
        out_specs=[ospec] * (2 * ng),
        out_shape=[jax.ShapeDtypeStruct((n_pad, d), jnp.float32)] * (2 * ng))(
            *ps, norm16)
    return list(outs[:ng]), list(outs[ng:])


def _tc_tag_matmul(feats, w_pad, b_pad, blk_off, n_rows):
    k1 = len(feats)
    d_in_p = feats[0].shape[1]
    d_out_p = w_pad.shape[2]
    BN = 1024

    def body(*refs):
        f_refs = refs[:k1]
        w_ref, b_ref, h_ref = refs[k1], refs[k1 + 1], refs[k1 + 2]
        acc = jnp.zeros((BN, d_out_p), jnp.float32)
        for k in range(k1):
            acc = acc + jnp.dot(f_refs[k][...], w_ref[k],
                                preferred_element_type=jnp.float32)
        y = jnp.maximum(acc + b_ref[...], 0.0)
        ss = jnp.sum(y * y, axis=1, keepdims=True)
        h_ref[...] = y / jnp.maximum(jnp.sqrt(ss), 1e-12)

    in_specs = [pl.BlockSpec((BN, d_in_p), lambda i, o=blk_off: (i + o, 0))
                for _ in range(k1)]
    in_specs += [pl.BlockSpec((k1, d_in_p, d_out_p), lambda i: (0, 0, 0)),
                 pl.BlockSpec((1, d_out_p), lambda i: (0, 0))]
    return pl.pallas_call(
        body, grid=(n_rows // BN,),
        in_specs=in_specs,
        out_specs=pl.BlockSpec((BN, d_out_p), lambda i: (i, 0)),
        out_shape=jax.ShapeDtypeStruct((n_rows, d_out_p), jnp.float32))(
            *feats, w_pad, b_pad)


def _tc_segment_max(h, ids, n_seg):
    n_pad, d = h.shape
    g_pad = (n_seg + 7) // 8 * 8

    def body(h_ref, id_ref, o_ref):
        def gbody(g, carry):
            m = id_ref[...] == g
            vals = jnp.where(m, h_ref[...], -jnp.inf)
            mx = jnp.max(vals, axis=0, keepdims=True)
            o_ref[pl.ds(g, 1), :] = jnp.where(jnp.isfinite(mx), mx, 0.0)
            return carry
        lax.fori_loop(0, n_seg, gbody, 0)

    return pl.pallas_call(
        body, out_shape=jax.ShapeDtypeStruct((g_pad, d), jnp.float32))(h, ids)


def _tc_attention(seq, mask, wqkv, bqkv, wproj, bproj):
    def body(x_ref, m_ref, wq_ref, bq_ref, wp_ref, bp_ref, o_ref):
        x = x_ref[...]
        qkv = jnp.dot(x, wq_ref[...], preferred_element_type=jnp.float32)
        qkv = qkv + bq_ref[...]
        q = qkv[:, :DIM_C]
        kk = qkv[:, DIM_C:2 * DIM_C]
        v = qkv[:, 2 * DIM_C:]
        a = lax.dot_general(q, kk, (((1,), (1,)), ((), ())),
                            preferred_element_type=jnp.float32)
        a = a * (DIM_C ** -0.5)
        a = jnp.where(m_ref[...] == 0.0, -1e9, a)
        a = a - jnp.max(a, axis=1, keepdims=True)
        e = jnp.exp(a)
        p = e / jnp.sum(e, axis=1, keepdims=True)
        o = jnp.dot(p, v, preferred_element_type=jnp.float32)
        o_ref[...] = jnp.dot(o, wp_ref[...],
                             preferred_element_type=jnp.float32) + bp_ref[...]

    return pl.pallas_call(
        body, out_shape=jax.ShapeDtypeStruct((SEQ_L, DIM_C), jnp.float32))(
            seq, mask, wqkv, bqkv.reshape(1, -1), wproj, bproj.reshape(1, -1))


def _tc_mlp1(x, w, b):
    kp, n_out = w.shape
    BK = 512

    def body(x_ref, w_ref, b_ref, o_ref):
        @pl.when(pl.program_id(0) == 0)
        def _init():
            o_ref[...] = jnp.zeros_like(o_ref)

        o_ref[...] += jnp.dot(x_ref[...], w_ref[...],
                              preferred_element_type=jnp.float32)

        @pl.when(pl.program_id(0) == pl.num_programs(0) - 1)
        def _fin():
            o_ref[...] = jnp.maximum(o_ref[...] + b_ref[...], 0.0)

    return pl.pallas_call(
        body, grid=(kp // BK,),
        in_specs=[pl.BlockSpec((1, BK), lambda i: (0, i)),
                  pl.BlockSpec((BK, n_out), lambda i: (i, 0)),
                  pl.BlockSpec((1, n_out), lambda i: (0, 0))],
        out_specs=pl.BlockSpec((1, n_out), lambda i: (0, 0)),
        out_shape=jax.ShapeDtypeStruct((1, n_out), jnp.float32))(x, w, b)


def _tc_mlp_rest(x, w2, b2, w3, b3, w4, b4):
    def body(x_ref, w2_ref, b2_ref, w3_ref, b3_ref, w4_ref, b4_ref, o_ref):
        h = jnp.dot(x_ref[...], w2_ref[...], preferred_element_type=jnp.float32)
        h = jnp.maximum(h + b2_ref[...], 0.0)
        h = jnp.dot(h, w3_ref[...], preferred_element_type=jnp.float32)
        h = jnp.maximum(h + b3_ref[...], 0.0)
        z = jnp.dot(h, w4_ref[...], preferred_element_type=jnp.float32)
        z = z + b4_ref[...]
        o_ref[...] = 1.0 / (1.0 + jnp.exp(-z))

    return pl.pallas_call(
        body, out_shape=jax.ShapeDtypeStruct((1, w4.shape[1]), jnp.float32))(
            x, w2, b2, w3, b3, w4, b4)


def _attn_mask_np():
    n = GL_G + GP_G
    m = np.eye(SEQ_L, dtype=np.float32)
    m[n:, :] = 0.0
    m[:, n:] = 0.0
    m[:, n - 1] = 1.0
    m[n - 1, :] = 1.0
    m[n - 1, n - 1] = 0.0
    return m


def _pad2(x, r, c):
    out = jnp.zeros((r, c), jnp.float32)
    return out.at[:x.shape[0], :x.shape[1]].set(x)


def _pad_w(w, b, d_in, d_in_p, d_out, d_out_p, ng=1):
    w_r = w.reshape(K_HOP + 1, d_in, d_out)
    w_pad = jnp.zeros((K_HOP + 1, ng * d_in_p, d_out_p), jnp.float32)
    w_pad = w_pad.at[:, :d_in, :d_out].set(w_r)
    w_pad = w_pad.reshape((K_HOP + 1) * ng, d_in_p, d_out_p)
    b_pad = jnp.zeros((1, d_out_p), jnp.float32).at[0, :d_out].set(b)
    return w_pad, b_pad


def _pad_edges(src, dst, e_pad, zero_row):
    npad = e_pad - src.shape[0]
    src_p = jnp.concatenate([src, jnp.full((npad,), zero_row, jnp.int32)])
    dst_p = jnp.concatenate([dst, jnp.full((npad,), zero_row, jnp.int32)])
    return src_p, dst_p.reshape(-1, CHUNK)


def kernel(x_protein, x_ligand, edge_index_protein, edge_index_ligand,
           graph_ids_protein, graph_ids_ligand, Wp1, bp1, Wp2, bp2,
           Wl1, bl1, Wl2, bl2, Wl3, bl3, Wqkv, bqkv, Wproj, bproj,
           Wf1, bf1, Wf2, bf2, Wf3, bf3, Wout, bout):
    srcm = jnp.concatenate([edge_index_protein[0],
                            edge_index_ligand[0] + NP_PAD])
    dstm = jnp.concatenate([edge_index_protein[1],
                            edge_index_ligand[1] + NP_PAD])
    srcm1, dstm2 = _pad_edges(srcm, dstm, EM_PAD, NM_PAD - 1)
    srcp1, dstp2 = _pad_edges(edge_index_protein[0], edge_index_protein[1],
                              EP_PAD, NP_PAD - 1)
    srcl1, dstl2 = _pad_edges(edge_index_ligand[0], edge_index_ligand[1],
                              EL_PAD, NL_PAD - 1)

    ones_tab = (jnp.zeros((NM_PAD, 16), jnp.float32)
                .at[:NP_N].set(1.0)
                .at[NP_PAD:NP_PAD + NL_N].set(1.0))
    degp = _sc_scatter_partials(ones_tab, srcm1, dstm2, NM_PAD, 16, 2, 4)
    norm16 = _tc_norm_from_deg(degp.reshape(2, NM_PAD, 16))
    normp16 = norm16[:NP_PAD]
    norml16 = norm16[NP_PAD:]

    xp = jnp.zeros((NP_PAD, D_IN), jnp.float32).at[:NP_N].set(x_protein)
    xl = jnp.zeros((NL_PAD, D_IN), jnp.float32).at[:NL_N].set(x_ligand)

    feats_p = _hops(xp, normp16, srcp1, dstp2, NP_PAD, 2, 2, True)
    feats_l = _hops(xl, norml16, srcl1, dstl2, NL_PAD, 2, 2, True)
    wpp, bpp = _pad_w(Wp1, bp1, 128, 64, 50, 64, ng=2)
    wlp, blp = _pad_w(Wl1, bl1, 128, 64, 50, 64, ng=2)
    hp = _tc_tag_matmul(feats_p, wpp, bpp, 0, NP_PAD)
    hl = _tc_tag_matmul(feats_l, wlp, blp, 0, NL_PAD)
    h = jnp.concatenate([hp, hl], axis=0)

    feats = _hops(h, norm16, srcm1, dstm2, NM_PAD, 2, 2, True)
    wpp, bpp = _pad_w(Wp2, bp2, 50, 32, 45, 48, ng=2)
    wlp, blp = _pad_w(Wl2, bl2, 50, 32, 45, 48, ng=2)
    hp = _tc_tag_matmul(feats, wpp, bpp, 0, NP_PAD)
    hl = _tc_tag_matmul(feats, wlp, blp, NP_PAD // 1024, NL_PAD)

    feats = _hops(hl, norml16, srcl1, dstl2, NL_PAD, 2, 2, False)
    wlp, blp = _pad_w(Wl3, bl3, 45, 48, 45, 48)
    hl = _tc_tag_matmul(feats, wlp, blp, 0, NL_PAD)

    ids_p = jnp.concatenate(
        [graph_ids_protein,
         jnp.full((NP_PAD - NP_N,), np.int32(10 ** 6), jnp.int32)])
    ids_l = jnp.concatenate(
        [graph_ids_ligand,
         jnp.full((NL_PAD - NL_N,), np.int32(10 ** 6), jnp.int32)])
    prot_rep = _tc_segment_max(hp, ids_p.reshape(NP_PAD, 1), GP_G)
    lig_rep = _tc_segment_max(hl, ids_l.reshape(NL_PAD, 1), GL_G)

    seq = jnp.concatenate(
        [lig_rep[:GL_G, :DIM_C], prot_rep[:GP_G, :DIM_C],
         jnp.zeros((SEQ_L - GL_G - GP_G, DIM_C), jnp.float32)], axis=0)
    mask = jnp.asarray(_attn_mask_np())
    att = _tc_attention(seq, mask, Wqkv, bqkv, Wproj, bproj)

    xh = att.reshape(1, SEQ_L * DIM_C)
    xh_p = _pad2(xh, 1, 7168)
    w1 = _pad2(Wf1, 7168, 2048)
    b1 = _pad2(bf1.reshape(1, -1), 1, 2048)
    h1 = _tc_mlp1(xh_p, w1, b1)

    w2 = _pad2(Wf2, 2048, 1024)
    b2 = _pad2(bf2.reshape(1, -1), 1, 1024)
    w3 = _pad2(Wf3, 1024, 512)
    b3 = _pad2(bf3.reshape(1, -1), 1, 512)
    w4 = _pad2(Wout, 512, 128)
    b4 = _pad2(bout.reshape(1, -1), 1, 128)
    out = _tc_mlp_rest(h1, w2, b2, w3, b3, w4, b4)
    return out[0:1, 0:1]


# speedup vs baseline: 1.9718x; 1.0105x over previous
import functools

import numpy as np
import jax
import jax.numpy as jnp
from jax import lax
from jax.experimental import pallas as pl
from jax.experimental.pallas import tpu as pltpu
from jax.experimental.pallas import tpu_sc as plsc

NP_N = 10000
EP_E = 320000
NL_N = 5000
EL_E = 160000
GP_G = 100
GL_G = 49
D_IN = 128
K_HOP = 8
SEQ_L = 150
DIM_C = 45

NCORE = 2
NSUB = 16
CHUNK = 128
ZROWS = 16

NP_PAD = 10240
NL_PAD = 5120
NM_PAD = NP_PAD + NL_PAD
EP_PAD = 327680
EL_PAD = 163840
EM_PAD = 491520


def _sc_scatter_partials(tab, src1d, dst2d, n_pad, d, nbuf, gb):
    chunks = dst2d.shape[0]
    nw = NCORE * NSUB
    cpt = chunks // nw
    ngroups = cpt // (nbuf * gb)
    rows_per_tile = n_pad // NSUB
    mesh = plsc.VectorSubcoreMesh(
        core_axis_name="c", subcore_axis_name="s",
        num_cores=NCORE, num_subcores=NSUB)

    @functools.partial(
        pl.kernel,
        out_type=jax.ShapeDtypeStruct((NCORE * n_pad, d), jnp.float32),
        mesh=mesh,
        compiler_params=pltpu.CompilerParams(use_tc_tiling_on_sc=False),
        scratch_types=[
            pltpu.VMEM((nbuf, gb * CHUNK), jnp.int32),
            pltpu.VMEM((nbuf, gb, CHUNK), jnp.int32),
            pltpu.VMEM((nbuf, gb * CHUNK, d), jnp.float32),
            pltpu.VMEM((ZROWS, d), jnp.float32),
            pltpu.VMEM_SHARED((n_pad, d), jnp.float32),
            pltpu.SemaphoreType.DMA,
            pltpu.SemaphoreType.DMA,
            pltpu.SemaphoreType.DMA,
        ],
    )
    def k(tab_hbm, src_hbm, dst_hbm, out_hbm,
          src_v, dst_v, bufs, zero_v, acc_sh, isem, gsem, ssem):
        cid = lax.axis_index("c")
        sid = lax.axis_index("s")
        tile = cid * NSUB + sid
        e0 = tile * cpt * CHUNK
        c0 = tile * cpt

        for i in range(ZROWS):
            for j in range(d // 16):
                zero_v[i, pl.ds(j * 16, 16)] = jnp.zeros((16,), jnp.float32)
        row0 = sid * rows_per_tile

        def zbody(i, carry):
            pltpu.async_copy(
                zero_v, acc_sh.at[pl.ds(row0 + i * ZROWS, ZROWS)], isem)
            return carry
        lax.fori_loop(0, rows_per_tile // ZROWS, zbody, 0)

        def zwait(i, carry):
            pltpu.make_async_copy(
                zero_v, acc_sh.at[pl.ds(row0 + i * ZROWS, ZROWS)], isem).wait()
            return carry
        lax.fori_loop(0, rows_per_tile // ZROWS, zwait, 0)
        plsc.subcore_barrier()

        def idx_copy(g, b, start):
            gc = (g * nbuf + b) * gb
            s_hbm = src_hbm.at[pl.ds(e0 + gc * CHUNK, gb * CHUNK)]
            d_hbm = dst_hbm.at[pl.ds(c0 + gc, gb)]
            if start:
                pltpu.async_copy(s_hbm, src_v.at[b], isem)
                pltpu.async_copy(d_hbm, dst_v.at[b], isem)
            else:
                pltpu.make_async_copy(s_hbm, src_v.at[b], isem).wait()
                pltpu.make_async_copy(d_hbm, dst_v.at[b], isem).wait()

        for b in range(nbuf):
            idx_copy(0, b, True)
        for b in range(nbuf):
            idx_copy(0, b, False)
            pltpu.async_copy(tab_hbm.at[src_v.at[b]], bufs.at[b], gsem)

        def gbody(g, carry):
            for b in range(nbuf):
                pltpu.make_async_copy(
                    tab_hbm.at[src_v.at[b]], bufs.at[b], gsem).wait()
                for j in range(gb):
                    pltpu.async_copy(
                        bufs.at[b, pl.ds(j * CHUNK, CHUNK)],
                        acc_sh.at[dst_v.at[b, j]], ssem, add=True)
            for b in range(nbuf):
                for j in range(gb):
                    pltpu.make_async_copy(
                        bufs.at[b, pl.ds(j * CHUNK, CHUNK)],
                        acc_sh.at[dst_v.at[b, j]], ssem).wait()

                @pl.when(g < ngroups - 1)
                def _regather(b=b):
                    idx_copy(g + 1, b, True)
                    idx_copy(g + 1, b, False)
                    pltpu.async_copy(
                        tab_hbm.at[src_v.at[b]], bufs.at[b], gsem)
            return carry
        lax.fori_loop(0, ngroups, gbody, 0)
        plsc.subcore_barrier()

        out0 = cid * n_pad + row0
        pltpu.sync_copy(acc_sh.at[pl.ds(row0, rows_per_tile)],
                        out_hbm.at[pl.ds(out0, rows_per_tile)])

    return k(tab, src1d, dst2d)


def _sc_tag_layer(h0, norm16, src1d, dst2d, n_pad, d, nbuf, gb):
    chunks = dst2d.shape[0]
    cpt = chunks // NSUB
    ngroups = cpt // (nbuf * gb)
    rows_per_tile = n_pad // NSUB
    rb = 64
    mesh = plsc.VectorSubcoreMesh(
        core_axis_name="c", subcore_axis_name="s",
        num_cores=NCORE, num_subcores=NSUB)

    @functools.partial(
        pl.kernel,
        out_type=[jax.ShapeDtypeStruct((NCORE * K_HOP * n_pad, d),
                                       jnp.float32),
                  jax.ShapeDtypeStruct((NCORE * n_pad, d), jnp.float32)],
        mesh=mesh,
        compiler_params=pltpu.CompilerParams(use_tc_tiling_on_sc=False),
        scratch_types=[
            pltpu.VMEM((nbuf, gb * CHUNK), jnp.int32),
            pltpu.VMEM((nbuf, gb, CHUNK), jnp.int32),
            pltpu.VMEM((nbuf, gb * CHUNK, d), jnp.float32),
            pltpu.VMEM((ZROWS, d), jnp.float32),
            pltpu.VMEM((rows_per_tile, 16), jnp.float32),
            pltpu.VMEM((rb, d), jnp.float32),
            pltpu.VMEM((rb, d), jnp.float32),
            pltpu.VMEM_SHARED((n_pad, d), jnp.float32),
            pltpu.SemaphoreType.DMA,
            pltpu.SemaphoreType.DMA,
            pltpu.SemaphoreType.DMA,
        ],
    )
    def k(h0_hbm, nrm_hbm, src_hbm, dst_hbm, feats_hbm, s2_hbm,
          src_v, dst_v, bufs, zero_v, nrm_v, abuf, hbuf, acc_sh,
          isem, gsem, ssem):
        cid = lax.axis_index("c")
        sid = lax.axis_index("s")
        e0 = sid * cpt * CHUNK
        c0 = sid * cpt
        row0 = sid * rows_per_tile
        soff = cid * n_pad

        pltpu.sync_copy(nrm_hbm.at[pl.ds(row0, rows_per_tile)], nrm_v)
        for i in range(ZROWS):
            for j in range(d // 16):
                zero_v[i, pl.ds(j * 16, 16)] = jnp.zeros((16,), jnp.float32)

        def scale_rows(i, out_h):
            def rowf(j, carry):
                nv = nrm_v[i * rb + j, pl.ds(0, 16)]
                for jv in range(d // 16):
                    a = abuf[j, pl.ds(jv * 16, 16)]
                    h = a * nv
                    if out_h:
                        abuf[j, pl.ds(jv * 16, 16)] = h
                    hbuf[j, pl.ds(jv * 16, 16)] = h * nv
                return carry
            lax.fori_loop(0, rb, rowf, 0)

        def p0(i, carry):
            pltpu.sync_copy(h0_hbm.at[pl.ds(soff + row0 + i * rb, rb)], abuf)
            scale_rows(i, False)
            pltpu.sync_copy(hbuf, s2_hbm.at[pl.ds(soff + row0 + i * rb, rb)])
            return carry
        lax.fori_loop(0, rows_per_tile // rb, p0, 0)
        plsc.subcore_barrier()

        def add_soff(b):
            for v in range(gb * CHUNK // 16):
                src_v[b, pl.ds(v * 16, 16)] = \
                    src_v[b, pl.ds(v * 16, 16)] + soff

        def idx_copy(g, b, start):
            gc = (g * nbuf + b) * gb
            s_hbm = src_hbm.at[pl.ds(e0 + gc * CHUNK, gb * CHUNK)]
            d_hbm = dst_hbm.at[pl.ds(c0 + gc, gb)]
            if start:
                pltpu.async_copy(s_hbm, src_v.at[b], isem)
                pltpu.async_copy(d_hbm, dst_v.at[b], isem)
            else:
                pltpu.make_async_copy(s_hbm, src_v.at[b], isem).wait()
                pltpu.make_async_copy(d_hbm, dst_v.at[b], isem).wait()

        def hop(kh, carry):
            def zbody(i, c2):
                pltpu.async_copy(
                    zero_v, acc_sh.at[pl.ds(row0 + i * ZROWS, ZROWS)], isem)
                return c2
            lax.fori_loop(0, rows_per_tile // ZROWS, zbody, 0)

            def zwait(i, c2):
                pltpu.make_async_copy(
                    zero_v,
                    acc_sh.at[pl.ds(row0 + i * ZROWS, ZROWS)], isem).wait()
                return c2
            lax.fori_loop(0, rows_per_tile // ZROWS, zwait, 0)
            plsc.subcore_barrier()

            for b in range(nbuf):
                idx_copy(0, b, True)
            for b in range(nbuf):
                idx_copy(0, b, False)
                add_soff(b)
                pltpu.async_copy(s2_hbm.at[src_v.at[b]], bufs.at[b], gsem)

            def gbody(g, c2):
                for b in range(nbuf):
                    pltpu.make_async_copy(
                        s2_hbm.at[src_v.at[b]], bufs.at[b], gsem).wait()
                    for j in range(gb):
                        pltpu.async_copy(
                            bufs.at[b, pl.ds(j * CHUNK, CHUNK)],
                            acc_sh.at[dst_v.at[b, j]], ssem, add=True)
                for b in range(nbuf):
                    for j in range(gb):
                        pltpu.make_async_copy(
                            bufs.at[b, pl.ds(j * CHUNK, CHUNK)],
                            acc_sh.at[dst_v.at[b, j]], ssem).wait()

                    @pl.when(g < ngroups - 1)
                    def _regather(b=b):
                        idx_copy(g + 1, b, True)
                        idx_copy(g + 1, b, False)
                        add_soff(b)
                        pltpu.async_copy(
                            s2_hbm.at[src_v.at[b]], bufs.at[b], gsem)
                return c2
            lax.fori_loop(0, ngroups, gbody, 0)
            plsc.subcore_barrier()

            def cb(i, c2):
                pltpu.sync_copy(acc_sh.at[pl.ds(row0 + i * rb, rb)], abuf)
                scale_rows(i, True)
                pltpu.sync_copy(
                    abuf,
                    feats_hbm.at[pl.ds((cid * K_HOP + kh) * n_pad
                                       + row0 + i * rb, rb)])
                pltpu.sync_copy(
                    hbuf, s2_hbm.at[pl.ds(soff + row0 + i * rb, rb)])
                return c2
            lax.fori_loop(0, rows_per_tile // rb, cb, 0)
            plsc.subcore_barrier()
            return carry
        lax.fori_loop(0, K_HOP, hop, 0)

    return k(h0, norm16, src1d, dst2d)[0]


def _hops(h0, norm16, src1d, dst2d, n_pad, nbuf, gb, split):
    d = h0.shape[1]
    if split:
        dh = d // 2
        h0_in = jnp.concatenate([h0[:, :dh], h0[:, dh:]], axis=0)
        groups0 = [h0[:, :dh], h0[:, dh:]]
    else:
        dh = d
        h0_in = jnp.concatenate([h0, h0], axis=0)
        groups0 = [h0]
    f = _sc_tag_layer(h0_in, norm16, src1d, dst2d, n_pad, dh, nbuf, gb)
    f4 = f.reshape(NCORE, K_HOP, n_pad, dh)
    feats = list(groups0)
    for kk in range(K_HOP):
        if split:
            feats.extend([f4[0, kk], f4[1, kk]])
        else:
            feats.append(f4[0, kk])
    return feats


def _tc_norm_from_deg(p):
    n_pad = p.shape[1]

    def body(p_ref, o_ref):
        deg = p_ref[0] + p_ref[1]
        o_ref[...] = lax.rsqrt(jnp.maximum(deg, 1.0))

    return pl.pallas_call(
        body, out_shape=jax.ShapeDtypeStruct((n_pad, 16), jnp.float32))(p)


def _tc_scale(xs, norm16):
    ng = len(xs)
    n_pad, d = xs[0].shape
    BN = 1024

    def body(*refs):
        nrm = refs[ng][...][:, :1]
        for gi in range(ng):
            refs[ng + 1 + gi][...] = refs[gi][...] * nrm

    spec = pl.BlockSpec((BN, d), lambda i: (i, 0))
    return pl.pallas_call(
        body, grid=(n_pad // BN,),
        in_specs=[spec] * ng + [pl.BlockSpec((BN, 16), lambda i: (i, 0))],
        out_specs=[spec] * ng,
        out_shape=[jax.ShapeDtypeStruct((n_pad, d), jnp.float32)] * ng)(
            *xs, norm16)


def _tc_combine(ps, norm16):
    ng = len(ps)
    _, n_pad, d = ps[0].shape
    BN = 1024

    def body(*refs):
        nrm = refs[ng][...][:, :1]
        for gi in range(ng):
            h = (refs[gi][0] + refs[gi][1]) * nrm
            refs[ng + 1 + gi][...] = h
            refs[ng + 1 + ng + gi][...] = h * nrm

    pspec = pl.BlockSpec((2, BN, d), lambda i: (0, i, 0))
    ospec = pl.BlockSpec((BN, d), lambda i: (i, 0))
    outs = pl.pallas_call(
        body, grid=(n_pad // BN,),
        in_specs=[pspec] * ng + [pl.BlockSpec((BN, 16), lambda i: (i, 0))],
        out_specs=[ospec] * (2 * ng),
        out_shape=[jax.ShapeDtypeStruct((n_pad, d), jnp.float32)] * (2 * ng))(
            *ps, norm16)
    return list(outs[:ng]), list(outs[ng:])


def _tc_tag_matmul(feats, w_pad, b_pad, blk_off, n_rows):
    k1 = len(feats)
    d_in_p = feats[0].shape[1]
    d_out_p = w_pad.shape[2]
    BN = 1024

    def body(*refs):
        f_refs = refs[:k1]
        w_ref, b_ref, h_ref = refs[k1], refs[k1 + 1], refs[k1 + 2]
        acc = jnp.zeros((BN, d_out_p), jnp.float32)
        for k in range(k1):
            acc = acc + jnp.dot(f_refs[k][...], w_ref[k],
                                preferred_element_type=jnp.float32)
        y = jnp.maximum(acc + b_ref[...], 0.0)
        ss = jnp.sum(y * y, axis=1, keepdims=True)
        h_ref[...] = y / jnp.maximum(jnp.sqrt(ss), 1e-12)

    in_specs = [pl.BlockSpec((BN, d_in_p), lambda i, o=blk_off: (i + o, 0))
                for _ in range(k1)]
    in_specs += [pl.BlockSpec((k1, d_in_p, d_out_p), lambda i: (0, 0, 0)),
                 pl.BlockSpec((1, d_out_p), lambda i: (0, 0))]
    return pl.pallas_call(
        body, grid=(n_rows // BN,),
        in_specs=in_specs,
        out_specs=pl.BlockSpec((BN, d_out_p), lambda i: (i, 0)),
        out_shape=jax.ShapeDtypeStruct((n_rows, d_out_p), jnp.float32))(
            *feats, w_pad, b_pad)


def _tc_segment_max(h, ids, n_seg):
    n_pad, d = h.shape
    g_pad = (n_seg + 7) // 8 * 8

    def body(h_ref, id_ref, o_ref):
        def gbody(g, carry):
            m = id_ref[...] == g
            vals = jnp.where(m, h_ref[...], -jnp.inf)
            mx = jnp.max(vals, axis=0, keepdims=True)
            o_ref[pl.ds(g, 1), :] = jnp.where(jnp.isfinite(mx), mx, 0.0)
            return carry
        lax.fori_loop(0, n_seg, gbody, 0)

    return pl.pallas_call(
        body, out_shape=jax.ShapeDtypeStruct((g_pad, d), jnp.float32))(h, ids)


def _tc_attention(seq, mask, wqkv, bqkv, wproj, bproj):
    def body(x_ref, m_ref, wq_ref, bq_ref, wp_ref, bp_ref, o_ref):
        x = x_ref[...]
        qkv = jnp.dot(x, wq_ref[...], preferred_element_type=jnp.float32)
        qkv = qkv + bq_ref[...]
        q = qkv[:, :DIM_C]
        kk = qkv[:, DIM_C:2 * DIM_C]
        v = qkv[:, 2 * DIM_C:]
        a = lax.dot_general(q, kk, (((1,), (1,)), ((), ())),
                            preferred_element_type=jnp.float32)
        a = a * (DIM_C ** -0.5)
        a = jnp.where(m_ref[...] == 0.0, -1e9, a)
        a = a - jnp.max(a, axis=1, keepdims=True)
        e = jnp.exp(a)
        p = e / jnp.sum(e, axis=1, keepdims=True)
        o = jnp.dot(p, v, preferred_element_type=jnp.float32)
        o_ref[...] = jnp.dot(o, wp_ref[...],
                             preferred_element_type=jnp.float32) + bp_ref[...]

    return pl.pallas_call(
        body, out_shape=jax.ShapeDtypeStruct((SEQ_L, DIM_C), jnp.float32))(
            seq, mask, wqkv, bqkv.reshape(1, -1), wproj, bproj.reshape(1, -1))


def _tc_mlp1(x, w, b):
    kp, n_out = w.shape
    BK = 512

    def body(x_ref, w_ref, b_ref, o_ref):
        @pl.when(pl.program_id(0) == 0)
        def _init():
            o_ref[...] = jnp.zeros_like(o_ref)

        o_ref[...] += jnp.dot(x_ref[...], w_ref[...],
                              preferred_element_type=jnp.float32)

        @pl.when(pl.program_id(0) == pl.num_programs(0) - 1)
        def _fin():
            o_ref[...] = jnp.maximum(o_ref[...] + b_ref[...], 0.0)

    return pl.pallas_call(
        body, grid=(kp // BK,),
        in_specs=[pl.BlockSpec((1, BK), lambda i: (0, i)),
                  pl.BlockSpec((BK, n_out), lambda i: (i, 0)),
                  pl.BlockSpec((1, n_out), lambda i: (0, 0))],
        out_specs=pl.BlockSpec((1, n_out), lambda i: (0, 0)),
        out_shape=jax.ShapeDtypeStruct((1, n_out), jnp.float32))(x, w, b)


def _tc_mlp_rest(x, w2, b2, w3, b3, w4, b4):
    def body(x_ref, w2_ref, b2_ref, w3_ref, b3_ref, w4_ref, b4_ref, o_ref):
        h = jnp.dot(x_ref[...], w2_ref[...], preferred_element_type=jnp.float32)
        h = jnp.maximum(h + b2_ref[...], 0.0)
        h = jnp.dot(h, w3_ref[...], preferred_element_type=jnp.float32)
        h = jnp.maximum(h + b3_ref[...], 0.0)
        z = jnp.dot(h, w4_ref[...], preferred_element_type=jnp.float32)
        z = z + b4_ref[...]
        o_ref[...] = 1.0 / (1.0 + jnp.exp(-z))

    return pl.pallas_call(
        body, out_shape=jax.ShapeDtypeStruct((1, w4.shape[1]), jnp.float32))(
            x, w2, b2, w3, b3, w4, b4)


def _attn_mask_np():
    n = GL_G + GP_G
    m = np.eye(SEQ_L, dtype=np.float32)
    m[n:, :] = 0.0
    m[:, n:] = 0.0
    m[:, n - 1] = 1.0
    m[n - 1, :] = 1.0
    m[n - 1, n - 1] = 0.0
    return m


def _pad2(x, r, c):
    out = jnp.zeros((r, c), jnp.float32)
    return out.at[:x.shape[0], :x.shape[1]].set(x)


def _pad_w(w, b, d_in, d_in_p, d_out, d_out_p, ng=1):
    w_r = w.reshape(K_HOP + 1, d_in, d_out)
    w_pad = jnp.zeros((K_HOP + 1, ng * d_in_p, d_out_p), jnp.float32)
    w_pad = w_pad.at[:, :d_in, :d_out].set(w_r)
    w_pad = w_pad.reshape((K_HOP + 1) * ng, d_in_p, d_out_p)
    b_pad = jnp.zeros((1, d_out_p), jnp.float32).at[0, :d_out].set(b)
    return w_pad, b_pad


def _pad_edges(src, dst, e_pad, zero_row):
    npad = e_pad - src.shape[0]
    src_p = jnp.concatenate([src, jnp.full((npad,), zero_row, jnp.int32)])
    dst_p = jnp.concatenate([dst, jnp.full((npad,), zero_row, jnp.int32)])
    return src_p, dst_p.reshape(-1, CHUNK)


def kernel(x_protein, x_ligand, edge_index_protein, edge_index_ligand,
           graph_ids_protein, graph_ids_ligand, Wp1, bp1, Wp2, bp2,
           Wl1, bl1, Wl2, bl2, Wl3, bl3, Wqkv, bqkv, Wproj, bproj,
           Wf1, bf1, Wf2, bf2, Wf3, bf3, Wout, bout):
    srcm = jnp.concatenate([edge_index_protein[0],
                            edge_index_ligand[0] + NP_PAD])
    dstm = jnp.concatenate([edge_index_protein[1],
                            edge_index_ligand[1] + NP_PAD])
    srcm1, dstm2 = _pad_edges(srcm, dstm, EM_PAD, NM_PAD - 1)
    srcp1, dstp2 = _pad_edges(edge_index_protein[0], edge_index_protein[1],
                              EP_PAD, NP_PAD - 1)
    srcl1, dstl2 = _pad_edges(edge_index_ligand[0], edge_index_ligand[1],
                              EL_PAD, NL_PAD - 1)

    ones_tab = (jnp.zeros((NM_PAD, 16), jnp.float32)
                .at[:NP_N].set(1.0)
                .at[NP_PAD:NP_PAD + NL_N].set(1.0))
    degp = _sc_scatter_partials(ones_tab, srcm1, dstm2, NM_PAD, 16, 2, 4)
    norm16 = _tc_norm_from_deg(degp.reshape(2, NM_PAD, 16))
    normp16 = norm16[:NP_PAD]
    norml16 = norm16[NP_PAD:]

    xp = jnp.zeros((NP_PAD, D_IN), jnp.float32).at[:NP_N].set(x_protein)
    xl = jnp.zeros((NL_PAD, D_IN), jnp.float32).at[:NL_N].set(x_ligand)

    feats_p = _hops(xp, normp16, srcp1, dstp2, NP_PAD, 2, 2, True)
    feats_l = _hops(xl, norml16, srcl1, dstl2, NL_PAD, 2, 2, True)
    wpp, bpp = _pad_w(Wp1, bp1, 128, 64, 50, 64, ng=2)
    wlp, blp = _pad_w(Wl1, bl1, 128, 64, 50, 64, ng=2)
    hp = _tc_tag_matmul(feats_p, wpp, bpp, 0, NP_PAD)
    hl = _tc_tag_matmul(feats_l, wlp, blp, 0, NL_PAD)
    h = jnp.concatenate([hp, hl], axis=0)

    feats = _hops(h, norm16, srcm1, dstm2, NM_PAD, 2, 2, True)
    wpp, bpp = _pad_w(Wp2, bp2, 50, 32, 45, 48, ng=2)
    wlp, blp = _pad_w(Wl2, bl2, 50, 32, 45, 48, ng=2)
    hp = _tc_tag_matmul(feats, wpp, bpp, 0, NP_PAD)
    hl = _tc_tag_matmul(feats, wlp, blp, NP_PAD // 1024, NL_PAD)

    feats = _hops(hl, norml16, srcl1, dstl2, NL_PAD, 2, 2, False)
    wlp, blp = _pad_w(Wl3, bl3, 45, 48, 45, 48)
    hl = _tc_tag_matmul(feats, wlp, blp, 0, NL_PAD)

    ids_p = jnp.concatenate(
        [graph_ids_protein,
         jnp.full((NP_PAD - NP_N,), np.int32(10 ** 6), jnp.int32)])
    ids_l = jnp.concatenate(
        [graph_ids_ligand,
         jnp.full((NL_PAD - NL_N,), np.int32(10 ** 6), jnp.int32)])
    prot_rep = _tc_segment_max(hp, ids_p.reshape(NP_PAD, 1), GP_G)
    lig_rep = _tc_segment_max(hl, ids_l.reshape(NL_PAD, 1), GL_G)

    seq = jnp.concatenate(
        [lig_rep[:GL_G, :DIM_C], prot_rep[:GP_G, :DIM_C],
         jnp.zeros((SEQ_L - GL_G - GP_G, DIM_C), jnp.float32)], axis=0)
    mask = jnp.asarray(_attn_mask_np())
    att = _tc_attention(seq, mask, Wqkv, bqkv, Wproj, bproj)

    xh = att.reshape(1, SEQ_L * DIM_C)
    xh_p = _pad2(xh, 1, 7168)
    w1 = _pad2(Wf1, 7168, 2048)
    b1 = _pad2(bf1.reshape(1, -1), 1, 2048)
    h1 = _tc_mlp1(xh_p, w1, b1)

    w2 = _pad2(Wf2, 2048, 1024)
    b2 = _pad2(bf2.reshape(1, -1), 1, 1024)
    w3 = _pad2(Wf3, 1024, 512)
    b3 = _pad2(bf3.reshape(1, -1), 1, 512)
    w4 = _pad2(Wout, 512, 128)
    b4 = _pad2(bout.reshape(1, -1), 1, 128)
    out = _tc_mlp_rest(h1, w2, b2, w3, b3, w4, b4)
    return out[0:1, 0:1]
